# baseline (speedup 1.0000x reference)
import functools

import jax
import jax.numpy as jnp
from jax import lax
from jax.experimental import pallas as pl
from jax.experimental.pallas import tpu as pltpu
from jax.experimental.pallas import tpu_sc as plsc

_EPS = 1e-5
_KS = 112
_KU = 96


def _gn(x, g, b):
    mu = jnp.mean(x, axis=1, keepdims=True)
    var = jnp.mean((x - mu) ** 2, axis=1, keepdims=True)
    return (x - mu) * jax.lax.rsqrt(var + _EPS) * g + b


def _pre_body(agts_ref, ctx_ref, w_qT_ref, g_q_ref, be_q_ref, wqT_ref,
              wcT_ref, w_agtT_ref, qh_ref, ch_ref, agt_lin_ref):
    agts = agts_ref[...]
    ctx = ctx_ref[...]
    q = jnp.dot(agts, w_qT_ref[...], preferred_element_type=jnp.float32)
    q = jax.nn.relu(_gn(q, g_q_ref[...], be_q_ref[...]))
    qh_ref[...] = jnp.dot(q, wqT_ref[...], preferred_element_type=jnp.float32)
    ch_ref[...] = jnp.dot(ctx, wcT_ref[...], preferred_element_type=jnp.float32)
    agt_lin_ref[...] = jnp.dot(agts, w_agtT_ref[...],
                               preferred_element_type=jnp.float32)


def _build_neighbors(ax, ay, cx, cy, th2_arr):
    N = ax.shape[0]
    KS, KU = _KS, _KU
    ncores, nsub, L = 2, 16, 16
    NW = ncores * nsub
    APW = N // NW
    SLOTS = APW * KS
    mesh = plsc.VectorSubcoreMesh(core_axis_name="c", subcore_axis_name="s")

    @functools.partial(
        pl.kernel, mesh=mesh,
        out_type=[jax.ShapeDtypeStruct((N * KS,), jnp.int32),
                  jax.ShapeDtypeStruct((N * KS,), jnp.float32),
                  jax.ShapeDtypeStruct((N * KS,), jnp.float32),
                  jax.ShapeDtypeStruct((N * KS,), jnp.float32)],
        scratch_types=[pltpu.VMEM((N,), jnp.float32),
                       pltpu.VMEM((N,), jnp.float32),
                       pltpu.VMEM((N + L,), jnp.float32),
                       pltpu.VMEM((N + L,), jnp.float32),
                       pltpu.VMEM((L,), jnp.float32),
                       pltpu.VMEM((SLOTS,), jnp.int32),
                       pltpu.VMEM((SLOTS,), jnp.float32),
                       pltpu.VMEM((SLOTS,), jnp.float32),
                       pltpu.VMEM((SLOTS,), jnp.float32)],
        compiler_params=pltpu.CompilerParams(needs_layout_passes=False),
    )
    def build(ax_h, ay_h, cx_h, cy_h, th2_h, jo_h, dxo_h, dyo_h, vo_h,
              cx_v, cy_v, ax_v, ay_v, th2_v, j_b, dx_b, dy_b, v_b):
        wid = lax.axis_index("s") * ncores + lax.axis_index("c")
        pltpu.sync_copy(cx_h, cx_v)
        pltpu.sync_copy(cy_h, cy_v)
        pltpu.sync_copy(ax_h, ax_v.at[pl.ds(0, N)])
        pltpu.sync_copy(ay_h, ay_v.at[pl.ds(0, N)])
        pltpu.sync_copy(th2_h, th2_v)
        th2v = th2_v[...]
        zf = jnp.zeros((L,), jnp.float32)
        zi = jnp.zeros((L,), jnp.int32)

        def zero_body(i, carry):
            j_b[pl.ds(i * L, L)] = zi
            dx_b[pl.ds(i * L, L)] = zf
            dy_b[pl.ds(i * L, L)] = zf
            return carry

        lax.fori_loop(0, SLOTS // L, zero_body, 0)
        lane = lax.iota(jnp.int32, L)

        def agent_body(a, carry):
            ag = wid * APW + a
            ax_s = ax_v[pl.ds(ag, L)][0]
            ay_s = ay_v[pl.ds(ag, L)][0]
            axv = jnp.full((L,), ax_s, jnp.float32)
            ayv = jnp.full((L,), ay_s, jnp.float32)
            sbase = a * KS

            def chunk_body(cb, ptr):
                cxc = cx_v[pl.ds(cb * L, L)]
                cyc = cy_v[pl.ds(cb * L, L)]
                dx = axv - cxc
                dy = ayv - cyc
                m = (dx * dx + dy * dy) <= th2v
                vals = jnp.full((L,), cb * L, jnp.int32) + lane
                off = sbase + jnp.minimum(ptr, KU)
                plsc.store_compressed(j_b.at[pl.ds(off, L)], vals, mask=m)
                plsc.store_compressed(dx_b.at[pl.ds(off, L)], dx, mask=m)
                plsc.store_compressed(dy_b.at[pl.ds(off, L)], dy, mask=m)
                pc = plsc.all_reduce_population_count(m)[0]
                return ptr + pc

            ptrf = lax.fori_loop(0, N // L, chunk_body, jnp.int32(0))
            cntv = jnp.full((L,), jnp.minimum(ptrf, KU), jnp.int32)
            for w in range(KS // L):
                base_w = jnp.full((L,), w * L, jnp.int32) + lane
                v_b[pl.ds(sbase + w * L, L)] = (base_w < cntv).astype(
                    jnp.float32)
            return carry

        lax.fori_loop(0, APW, agent_body, 0)
        out0 = wid * SLOTS
        pltpu.sync_copy(j_b, jo_h.at[pl.ds(out0, SLOTS)])
        pltpu.sync_copy(dx_b, dxo_h.at[pl.ds(out0, SLOTS)])
        pltpu.sync_copy(dy_b, dyo_h.at[pl.ds(out0, SLOTS)])
        pltpu.sync_copy(v_b, vo_h.at[pl.ds(out0, SLOTS)])

    return build(ax, ay, cx, cy, th2_arr)


def _gather_rows(tbl, idx):
    NT, D = tbl.shape
    B = idx.shape[0]
    ncores = 2
    NW = ncores * 16
    bpw = B // NW
    CH = 512
    NCH = bpw // CH
    mesh = plsc.VectorSubcoreMesh(core_axis_name="c", subcore_axis_name="s")

    @functools.partial(
        pl.kernel, mesh=mesh,
        out_type=jax.ShapeDtypeStruct((B, D), jnp.float32),
        scratch_types=[pltpu.VMEM((bpw,), jnp.int32),
                       pltpu.VMEM((CH, D), jnp.float32),
                       pltpu.SemaphoreType.DMA],
        compiler_params=pltpu.CompilerParams(needs_layout_passes=False),
    )
    def gk(tbl_h, idx_h, out_h, idx_v, rows_v, sem):
        wid = lax.axis_index("s") * ncores + lax.axis_index("c")
        base = wid * bpw
        pltpu.sync_copy(idx_h.at[pl.ds(base, bpw)], idx_v)

        def body(t, carry):
            off = t * CH
            pltpu.async_copy(tbl_h.at[idx_v.at[pl.ds(off, CH)]], rows_v,
                             sem).wait()
            pltpu.sync_copy(rows_v, out_h.at[pl.ds(base + off, CH)])
            return carry

        lax.fori_loop(0, NCH, body, 0)

    return gk(tbl, idx)


def _main_body(dx_ref, dy_ref, v_ref, chg_ref, qh_ref, w1x_ref, w1y_ref,
               b1_ref, w_dist2T_ref, g_dist_ref, be_dist_ref, wdT_ref,
               g_c1_ref, be_c1_ref, acc_ref):
    R = dx_ref.shape[0]
    BA = qh_ref.shape[0]
    K = R // BA
    dx = dx_ref[...]
    dy = dy_ref[...]
    d1 = jax.nn.relu(dx * w1x_ref[...] + dy * w1y_ref[...] + b1_ref[...])
    t = jnp.dot(d1, w_dist2T_ref[...], preferred_element_type=jnp.float32)
    d2 = jax.nn.relu(_gn(t, g_dist_ref[...], be_dist_ref[...]))
    seg = lax.broadcasted_iota(jnp.int32, (R, BA), 0) // K
    col = lax.broadcasted_iota(jnp.int32, (R, BA), 1)
    ind = (seg == col).astype(jnp.float32)
    qrows = jnp.dot(ind, qh_ref[...], preferred_element_type=jnp.float32)
    h = jnp.dot(d2, wdT_ref[...], preferred_element_type=jnp.float32)
    h = h + chg_ref[...] + qrows
    c = jax.nn.relu(_gn(h, g_c1_ref[...], be_c1_ref[...]))
    sel = ind * v_ref[...]
    acc_ref[...] = lax.dot_general(sel, c, (((0,), (0,)), ((), ())),
                                   preferred_element_type=jnp.float32)


def _epi_body(acc_ref, agt_lin_ref, agts_ref, w_c2T_ref, g_n_ref, be_n_ref,
              w_linT_ref, g_lin_ref, be_lin_ref, out_ref):
    msg = jnp.dot(acc_ref[...], w_c2T_ref[...],
                  preferred_element_type=jnp.float32)
    a = agt_lin_ref[...] + msg
    a = jax.nn.relu(_gn(a, g_n_ref[...], be_n_ref[...]))
    a = _gn(jnp.dot(a, w_linT_ref[...], preferred_element_type=jnp.float32),
            g_lin_ref[...], be_lin_ref[...])
    a = a + agts_ref[...]
    out_ref[...] = jax.nn.relu(a)


def kernel(agts, ctx, agt_ctrs, ctx_ctrs, w_dist1, b_dist1, w_dist2, g_dist,
           be_dist, w_q, g_q, be_q, w_c1, g_c1, be_c1, w_c2, w_agt, g_n, be_n,
           w_lin, g_lin, be_lin, dist_th):
    N, D = agts.shape
    K = _KS
    f32 = jnp.float32

    wdT = w_c1[:, :D].T
    wqT = w_c1[:, D:2 * D].T
    wcT = w_c1[:, 2 * D:].T
    w_qT = w_q.T
    w_agtT = w_agt.T
    w_dist2T = w_dist2.T
    w_c2T = w_c2.T
    w_linT = w_lin.T
    w1x = w_dist1[:, 0].reshape(1, D)
    w1y = w_dist1[:, 1].reshape(1, D)
    b1 = b_dist1.reshape(1, D)
    row = lambda v: v.reshape(1, D)
    th = jnp.asarray(dist_th, f32)
    th2_arr = jnp.full((16,), th * th, f32)
    ax = agt_ctrs[:, 0].astype(f32)
    ay = agt_ctrs[:, 1].astype(f32)
    cx = ctx_ctrs[:, 0].astype(f32)
    cy = ctx_ctrs[:, 1].astype(f32)

    BP = min(256, N)
    full = lambda shp: pl.BlockSpec(shp, lambda i: (0, 0))
    blk = lambda r: pl.BlockSpec((r, D), lambda i: (i, 0))
    qh, ch, agt_lin = pl.pallas_call(
        _pre_body,
        grid=(N // BP,),
        in_specs=[blk(BP), blk(BP), full((D, D)), full((1, D)), full((1, D)),
                  full((D, D)), full((D, D)), full((D, D))],
        out_specs=[blk(BP), blk(BP), blk(BP)],
        out_shape=[jax.ShapeDtypeStruct((N, D), f32)] * 3,
    )(agts, ctx, w_qT, row(g_q), row(be_q), wqT, wcT, w_agtT)

    nbr_j, nbr_dx, nbr_dy, nbr_v = _build_neighbors(ax, ay, cx, cy, th2_arr)

    chg = _gather_rows(ch, nbr_j)

    BA = 8
    RR = BA * K
    col1 = lambda: pl.BlockSpec((RR, 1), lambda i: (i, 0))
    acc = pl.pallas_call(
        _main_body,
        grid=(N // BA,),
        in_specs=[
            col1(), col1(), col1(),
            pl.BlockSpec((RR, D), lambda i: (i, 0)),
            pl.BlockSpec((BA, D), lambda i: (i, 0)),
            full((1, D)), full((1, D)), full((1, D)), full((D, D)),
            full((1, D)), full((1, D)), full((D, D)), full((1, D)),
            full((1, D)),
        ],
        out_specs=pl.BlockSpec((BA, D), lambda i: (i, 0)),
        out_shape=jax.ShapeDtypeStruct((N, D), f32),
    )(nbr_dx.reshape(N * K, 1), nbr_dy.reshape(N * K, 1),
      nbr_v.reshape(N * K, 1), chg, qh, w1x, w1y, b1, w_dist2T,
      row(g_dist), row(be_dist), wdT, row(g_c1), row(be_c1))

    out = pl.pallas_call(
        _epi_body,
        grid=(N // BP,),
        in_specs=[blk(BP), blk(BP), blk(BP), full((D, D)), full((1, D)),
                  full((1, D)), full((D, D)), full((1, D)), full((1, D))],
        out_specs=blk(BP),
        out_shape=jax.ShapeDtypeStruct((N, D), f32),
    )(acc, agt_lin, agts, w_c2T, row(g_n), row(be_n), w_linT, row(g_lin),
      row(be_lin))
    return out


# speedup vs baseline: 9.6895x; 9.6895x over previous
import functools

import jax
import jax.numpy as jnp
from jax import lax
from jax.experimental import pallas as pl
from jax.experimental.pallas import tpu as pltpu
from jax.experimental.pallas import tpu_sc as plsc

_EPS = 1e-5
_KS = 112
_KU = 96


def _gn(x, g, b):
    mu = jnp.mean(x, axis=1, keepdims=True)
    var = jnp.mean((x - mu) ** 2, axis=1, keepdims=True)
    return (x - mu) * jax.lax.rsqrt(var + _EPS) * g + b


def _pre_body(agts_ref, ctx_ref, w_qT_ref, g_q_ref, be_q_ref, wqT_ref,
              wcT_ref, w_agtT_ref, qh_ref, ch_ref, agt_lin_ref):
    agts = agts_ref[...]
    ctx = ctx_ref[...]
    q = jnp.dot(agts, w_qT_ref[...], preferred_element_type=jnp.float32)
    q = jax.nn.relu(_gn(q, g_q_ref[...], be_q_ref[...]))
    qh_ref[...] = jnp.dot(q, wqT_ref[...], preferred_element_type=jnp.float32)
    ch_ref[...] = jnp.dot(ctx, wcT_ref[...], preferred_element_type=jnp.float32)
    agt_lin_ref[...] = jnp.dot(agts, w_agtT_ref[...],
                               preferred_element_type=jnp.float32)


def _build_neighbors(ax, ay, cx, cy, th2_arr):
    N = ax.shape[0]
    KS, KU = _KS, _KU
    ncores, nsub, L = 2, 16, 16
    NW = ncores * nsub
    APW = N // NW
    SLOTS = APW * KS
    mesh = plsc.VectorSubcoreMesh(core_axis_name="c", subcore_axis_name="s")

    @functools.partial(
        pl.kernel, mesh=mesh,
        out_type=[jax.ShapeDtypeStruct((N * KS,), jnp.int32),
                  jax.ShapeDtypeStruct((N * KS,), jnp.float32),
                  jax.ShapeDtypeStruct((N * KS,), jnp.float32),
                  jax.ShapeDtypeStruct((N * KS,), jnp.float32)],
        scratch_types=[pltpu.VMEM((N,), jnp.float32),
                       pltpu.VMEM((N,), jnp.float32),
                       pltpu.VMEM((N + L,), jnp.float32),
                       pltpu.VMEM((N + L,), jnp.float32),
                       pltpu.VMEM((L,), jnp.float32),
                       pltpu.VMEM((SLOTS,), jnp.int32),
                       pltpu.VMEM((SLOTS,), jnp.float32),
                       pltpu.VMEM((SLOTS,), jnp.float32),
                       pltpu.VMEM((SLOTS,), jnp.float32)],
        compiler_params=pltpu.CompilerParams(needs_layout_passes=False),
    )
    def build(ax_h, ay_h, cx_h, cy_h, th2_h, jo_h, dxo_h, dyo_h, vo_h,
              cx_v, cy_v, ax_v, ay_v, th2_v, j_b, dx_b, dy_b, v_b):
        wid = lax.axis_index("s") * ncores + lax.axis_index("c")
        pltpu.sync_copy(cx_h, cx_v)
        pltpu.sync_copy(cy_h, cy_v)
        pltpu.sync_copy(ax_h, ax_v.at[pl.ds(0, N)])
        pltpu.sync_copy(ay_h, ay_v.at[pl.ds(0, N)])
        pltpu.sync_copy(th2_h, th2_v)
        th2v = th2_v[...]
        zf = jnp.zeros((L,), jnp.float32)
        zi = jnp.zeros((L,), jnp.int32)

        def zero_body(i, carry):
            j_b[pl.ds(i * L, L)] = zi
            dx_b[pl.ds(i * L, L)] = zf
            dy_b[pl.ds(i * L, L)] = zf
            return carry

        lax.fori_loop(0, SLOTS // L, zero_body, 0)
        lane = lax.iota(jnp.int32, L)

        def agent_body(a, carry):
            ag = wid * APW + a
            ax_s = ax_v[pl.ds(ag, L)][0]
            ay_s = ay_v[pl.ds(ag, L)][0]
            axv = jnp.full((L,), ax_s, jnp.float32)
            ayv = jnp.full((L,), ay_s, jnp.float32)
            sbase = a * KS

            def chunk_body(cb, ptr):
                cxc = cx_v[pl.ds(cb * L, L)]
                cyc = cy_v[pl.ds(cb * L, L)]
                dx = axv - cxc
                dy = ayv - cyc
                m = (dx * dx + dy * dy) <= th2v
                vals = jnp.full((L,), cb * L, jnp.int32) + lane
                off = sbase + jnp.minimum(ptr, KU)
                plsc.store_compressed(j_b.at[pl.ds(off, L)], vals, mask=m)
                plsc.store_compressed(dx_b.at[pl.ds(off, L)], dx, mask=m)
                plsc.store_compressed(dy_b.at[pl.ds(off, L)], dy, mask=m)
                pc = plsc.all_reduce_population_count(m)[0]
                return ptr + pc

            ptrf = lax.fori_loop(0, N // L, chunk_body, jnp.int32(0))
            cntv = jnp.full((L,), jnp.minimum(ptrf, KU), jnp.int32)
            for w in range(KS // L):
                base_w = jnp.full((L,), w * L, jnp.int32) + lane
                v_b[pl.ds(sbase + w * L, L)] = (base_w < cntv).astype(
                    jnp.float32)
            return carry

        lax.fori_loop(0, APW, agent_body, 0)
        out0 = wid * SLOTS
        pltpu.sync_copy(j_b, jo_h.at[pl.ds(out0, SLOTS)])
        pltpu.sync_copy(dx_b, dxo_h.at[pl.ds(out0, SLOTS)])
        pltpu.sync_copy(dy_b, dyo_h.at[pl.ds(out0, SLOTS)])
        pltpu.sync_copy(v_b, vo_h.at[pl.ds(out0, SLOTS)])

    return build(ax, ay, cx, cy, th2_arr)


def _gather_rows(tbl, idx):
    NT, D = tbl.shape
    B = idx.shape[0]
    ncores = 2
    NW = ncores * 16
    bpw = B // NW
    CH = 512
    NCH = bpw // CH
    mesh = plsc.VectorSubcoreMesh(core_axis_name="c", subcore_axis_name="s")

    nsub = 16
    rps = NT // nsub

    @functools.partial(
        pl.kernel, mesh=mesh,
        out_type=jax.ShapeDtypeStruct((B, D), jnp.float32),
        scratch_types=[pltpu.VMEM((bpw,), jnp.int32),
                       pltpu.VMEM((CH, D), jnp.float32),
                       pltpu.VMEM_SHARED((NT, D), jnp.float32),
                       pltpu.SemaphoreType.DMA],
        compiler_params=pltpu.CompilerParams(needs_layout_passes=False),
    )
    def gk(tbl_h, idx_h, out_h, idx_v, rows_v, tbl_sh, sem):
        wid = lax.axis_index("s") * ncores + lax.axis_index("c")
        sid = lax.axis_index("s")
        base = wid * bpw
        pltpu.sync_copy(tbl_h.at[pl.ds(sid * rps, rps)],
                        tbl_sh.at[pl.ds(sid * rps, rps)])
        pltpu.sync_copy(idx_h.at[pl.ds(base, bpw)], idx_v)
        plsc.subcore_barrier()

        def body(t, carry):
            off = t * CH
            pltpu.async_copy(tbl_sh.at[idx_v.at[pl.ds(off, CH)]], rows_v,
                             sem).wait()
            pltpu.sync_copy(rows_v, out_h.at[pl.ds(base + off, CH)])
            return carry

        lax.fori_loop(0, NCH, body, 0)

    return gk(tbl, idx)


def _main_body(dx_ref, dy_ref, v_ref, chg_ref, qh_ref, w1x_ref, w1y_ref,
               b1_ref, w_dist2T_ref, g_dist_ref, be_dist_ref, wdT_ref,
               g_c1_ref, be_c1_ref, acc_ref):
    R = dx_ref.shape[0]
    BA = qh_ref.shape[0]
    K = R // BA
    dx = dx_ref[...]
    dy = dy_ref[...]
    d1 = jax.nn.relu(dx * w1x_ref[...] + dy * w1y_ref[...] + b1_ref[...])
    t = jnp.dot(d1, w_dist2T_ref[...], preferred_element_type=jnp.float32)
    d2 = jax.nn.relu(_gn(t, g_dist_ref[...], be_dist_ref[...]))
    seg = lax.broadcasted_iota(jnp.int32, (R, BA), 0) // K
    col = lax.broadcasted_iota(jnp.int32, (R, BA), 1)
    ind = (seg == col).astype(jnp.float32)
    qrows = jnp.dot(ind, qh_ref[...], preferred_element_type=jnp.float32)
    h = jnp.dot(d2, wdT_ref[...], preferred_element_type=jnp.float32)
    h = h + chg_ref[...] + qrows
    c = jax.nn.relu(_gn(h, g_c1_ref[...], be_c1_ref[...]))
    sel = ind * v_ref[...]
    acc_ref[...] = lax.dot_general(sel, c, (((0,), (0,)), ((), ())),
                                   preferred_element_type=jnp.float32)


def _epi_body(acc_ref, agt_lin_ref, agts_ref, w_c2T_ref, g_n_ref, be_n_ref,
              w_linT_ref, g_lin_ref, be_lin_ref, out_ref):
    msg = jnp.dot(acc_ref[...], w_c2T_ref[...],
                  preferred_element_type=jnp.float32)
    a = agt_lin_ref[...] + msg
    a = jax.nn.relu(_gn(a, g_n_ref[...], be_n_ref[...]))
    a = _gn(jnp.dot(a, w_linT_ref[...], preferred_element_type=jnp.float32),
            g_lin_ref[...], be_lin_ref[...])
    a = a + agts_ref[...]
    out_ref[...] = jax.nn.relu(a)


def kernel(agts, ctx, agt_ctrs, ctx_ctrs, w_dist1, b_dist1, w_dist2, g_dist,
           be_dist, w_q, g_q, be_q, w_c1, g_c1, be_c1, w_c2, w_agt, g_n, be_n,
           w_lin, g_lin, be_lin, dist_th):
    N, D = agts.shape
    K = _KS
    f32 = jnp.float32

    wdT = w_c1[:, :D].T
    wqT = w_c1[:, D:2 * D].T
    wcT = w_c1[:, 2 * D:].T
    w_qT = w_q.T
    w_agtT = w_agt.T
    w_dist2T = w_dist2.T
    w_c2T = w_c2.T
    w_linT = w_lin.T
    w1x = w_dist1[:, 0].reshape(1, D)
    w1y = w_dist1[:, 1].reshape(1, D)
    b1 = b_dist1.reshape(1, D)
    row = lambda v: v.reshape(1, D)
    th = jnp.asarray(dist_th, f32)
    th2_arr = jnp.full((16,), th * th, f32)
    ax = agt_ctrs[:, 0].astype(f32)
    ay = agt_ctrs[:, 1].astype(f32)
    cx = ctx_ctrs[:, 0].astype(f32)
    cy = ctx_ctrs[:, 1].astype(f32)

    BP = min(256, N)
    full = lambda shp: pl.BlockSpec(shp, lambda i: (0, 0))
    blk = lambda r: pl.BlockSpec((r, D), lambda i: (i, 0))
    qh, ch, agt_lin = pl.pallas_call(
        _pre_body,
        grid=(N // BP,),
        in_specs=[blk(BP), blk(BP), full((D, D)), full((1, D)), full((1, D)),
                  full((D, D)), full((D, D)), full((D, D))],
        out_specs=[blk(BP), blk(BP), blk(BP)],
        out_shape=[jax.ShapeDtypeStruct((N, D), f32)] * 3,
    )(agts, ctx, w_qT, row(g_q), row(be_q), wqT, wcT, w_agtT)

    nbr_j, nbr_dx, nbr_dy, nbr_v = _build_neighbors(ax, ay, cx, cy, th2_arr)

    chg = _gather_rows(ch, nbr_j)

    BA = 8
    RR = BA * K
    col1 = lambda: pl.BlockSpec((RR, 1), lambda i: (i, 0))
    acc = pl.pallas_call(
        _main_body,
        grid=(N // BA,),
        in_specs=[
            col1(), col1(), col1(),
            pl.BlockSpec((RR, D), lambda i: (i, 0)),
            pl.BlockSpec((BA, D), lambda i: (i, 0)),
            full((1, D)), full((1, D)), full((1, D)), full((D, D)),
            full((1, D)), full((1, D)), full((D, D)), full((1, D)),
            full((1, D)),
        ],
        out_specs=pl.BlockSpec((BA, D), lambda i: (i, 0)),
        out_shape=jax.ShapeDtypeStruct((N, D), f32),
    )(nbr_dx.reshape(N * K, 1), nbr_dy.reshape(N * K, 1),
      nbr_v.reshape(N * K, 1), chg, qh, w1x, w1y, b1, w_dist2T,
      row(g_dist), row(be_dist), wdT, row(g_c1), row(be_c1))

    out = pl.pallas_call(
        _epi_body,
        grid=(N // BP,),
        in_specs=[blk(BP), blk(BP), blk(BP), full((D, D)), full((1, D)),
                  full((1, D)), full((D, D)), full((1, D)), full((1, D))],
        out_specs=blk(BP),
        out_shape=jax.ShapeDtypeStruct((N, D), f32),
    )(acc, agt_lin, agts, w_c2T, row(g_n), row(be_n), w_linT, row(g_lin),
      row(be_lin))
    return out
